# Initial kernel scaffold; baseline (speedup 1.0000x reference)
import jax
import jax.numpy as jnp
from jax.experimental import pallas as pl


def kernel(past_p, cur_p, batch_size, step):
    raise NotImplementedError("write your pallas kernel here")


# speedup vs baseline: 1.7172x; 1.7172x over previous
import functools

import jax
import jax.numpy as jnp
from jax.experimental import pallas as pl
from jax.experimental.pallas import tpu as pltpu

BEAMS = 8
VOCAB = 100000
CHUNK = 512
SUPER = 4096
NSUPER = 25
CPS = SUPER // CHUNK
NCHUNK = NSUPER * CPS
NEG_INF = float("-inf")
BIG_I32 = 2**31 - 1


def _k1_chunk_max(past_ref, cur_ref, out_ref):
    i = pl.program_id(0)
    x = cur_ref[...] + past_ref[...]
    col0 = i * SUPER
    cols = col0 + jax.lax.broadcasted_iota(jnp.int32, x.shape, 1)
    x = jnp.where(cols < VOCAB, x, NEG_INF)
    parts = [
        jnp.max(x[:, j * CHUNK:(j + 1) * CHUNK], axis=1, keepdims=True)
        for j in range(CPS)
    ]
    out_ref[...] = jnp.concatenate(parts, axis=1).reshape(1, x.shape[0], CPS)


def _k2_select(maxima_ref, rows_ref, cols_ref):
    vals = maxima_ref[...]
    groups = vals.shape[0]
    b = jax.lax.broadcasted_iota(jnp.int32, vals.shape, 1)
    c = jax.lax.broadcasted_iota(jnp.int32, vals.shape, 2)
    pid = b * NCHUNK + c
    g = jax.lax.broadcasted_iota(jnp.int32, (groups, 1, BEAMS), 0)
    row_cols = []
    col_cols = []
    for _ in range(BEAMS):
        m = jnp.max(vals, axis=(1, 2), keepdims=True)
        cand = jnp.where(vals == m, pid, BIG_I32)
        sel = jnp.min(cand, axis=(1, 2), keepdims=True)
        row_cols.append(sel // NCHUNK)
        col_cols.append(sel % NCHUNK)
        vals = jnp.where(pid == sel, NEG_INF, vals)
    beam = jnp.concatenate(row_cols, axis=2)
    rows_ref[...] = g * BEAMS + beam
    cols_ref[...] = jnp.concatenate(col_cols, axis=2)


def _k3_final(rows_ref, cols_ref, past_ref, *refs):
    cur_refs = refs[:BEAMS]
    topv_ref, sym_ref = refs[BEAMS], refs[BEAMS + 1]
    g = pl.program_id(0)

    vals_rows = []
    gidx_rows = []
    iota = jax.lax.broadcasted_iota(jnp.int32, (1, CHUNK), 1)
    for j in range(BEAMS):
        row = rows_ref[g, 0, j]
        colchunk = cols_ref[g, 0, j]
        bias = past_ref[row]
        x = cur_refs[j][...].reshape(1, CHUNK) + bias
        col = colchunk * CHUNK + iota
        valid = col < VOCAB
        beam = row - g * BEAMS
        gindex = beam * VOCAB + col
        vals_rows.append(jnp.where(valid, x, NEG_INF))
        gidx_rows.append(jnp.where(valid, gindex, BIG_I32))
    vals = jnp.concatenate(vals_rows, axis=0)
    gidx = jnp.concatenate(gidx_rows, axis=0)

    tv = []
    ts = []
    for _ in range(BEAMS):
        m = jnp.max(vals)
        sel = jnp.min(jnp.where(vals == m, gidx, BIG_I32))
        tv.append(m)
        ts.append(sel)
        vals = jnp.where(gidx == sel, NEG_INF, vals)
    topv_ref[...] = jnp.stack(tv).reshape(1, 1, BEAMS)
    sym_ref[...] = (jnp.stack(ts) % VOCAB).reshape(1, 1, BEAMS)


@jax.jit
def _run(past_p, cur_p):
    nrows = cur_p.shape[0]
    groups = nrows // BEAMS

    maxima3 = pl.pallas_call(
        _k1_chunk_max,
        grid=(NSUPER,),
        in_specs=[
            pl.BlockSpec((nrows, 1), lambda i: (0, 0)),
            pl.BlockSpec((nrows, SUPER), lambda i: (0, i)),
        ],
        out_specs=pl.BlockSpec((1, nrows, CPS), lambda i: (i, 0, 0)),
        out_shape=jax.ShapeDtypeStruct((NSUPER, nrows, CPS), jnp.float32),
        compiler_params=pltpu.CompilerParams(
            dimension_semantics=("arbitrary",),
        ),
    )(past_p, cur_p)

    maxima = jnp.transpose(maxima3, (1, 0, 2)).reshape(groups, BEAMS, NCHUNK)

    rows, cols = pl.pallas_call(
        _k2_select,
        out_shape=(
            jax.ShapeDtypeStruct((groups, 1, BEAMS), jnp.int32),
            jax.ShapeDtypeStruct((groups, 1, BEAMS), jnp.int32),
        ),
    )(maxima)

    grid_spec = pltpu.PrefetchScalarGridSpec(
        num_scalar_prefetch=3,
        grid=(groups,),
        in_specs=[
            pl.BlockSpec(
                (1, 1, CHUNK),
                (lambda g, rows_ref, cols_ref, past_ref, j=j:
                 (rows_ref[g, 0, j], 0, cols_ref[g, 0, j])),
            )
            for j in range(BEAMS)
        ],
        out_specs=[
            pl.BlockSpec((1, 1, BEAMS), lambda g, *_: (g, 0, 0)),
            pl.BlockSpec((1, 1, BEAMS), lambda g, *_: (g, 0, 0)),
        ],
    )

    topv, sym = pl.pallas_call(
        _k3_final,
        grid_spec=grid_spec,
        out_shape=(
            jax.ShapeDtypeStruct((groups, 1, BEAMS), jnp.float32),
            jax.ShapeDtypeStruct((groups, 1, BEAMS), jnp.int32),
        ),
        compiler_params=pltpu.CompilerParams(
            dimension_semantics=("arbitrary",),
        ),
    )(rows, cols, past_p.reshape(-1), *([cur_p.reshape(nrows, 1, VOCAB)] * BEAMS))

    return topv.reshape(-1, 1), sym.reshape(groups, BEAMS)


def kernel(past_p, cur_p, batch_size, step):
    del batch_size, step
    return _run(past_p, cur_p)
